# baseline (speedup 1.0000x reference)
import functools

import numpy as np
import jax
import jax.numpy as jnp
from jax import lax
from jax.experimental import pallas as pl
from jax.experimental.pallas import tpu as pltpu
from jax.experimental.pallas import tpu_sc as plsc

NC = 2
NS = 16
NW = NC * NS
CK = 96

ALPHA_C = 0.1
THETA_C = 0.5
SPLIT0 = 0.5


def _pad_up(v, m):
    return (v + m - 1) // m * m


def _make_sc_deg(np_, ch):
    mesh = plsc.VectorSubcoreMesh(core_axis_name="c", subcore_axis_name="s")
    rows_t = np_ // NS

    @functools.partial(
        pl.kernel,
        out_type=jax.ShapeDtypeStruct((NC * np_,), jnp.float32),
        mesh=mesh,
        scratch_types=[
            pltpu.VMEM((ch, CK), jnp.int32),
            pltpu.VMEM((CK,), jnp.float32),
            pltpu.VMEM((rows_t,), jnp.float32),
            pltpu.VMEM_SHARED((np_,), jnp.float32),
        ],
    )
    def deg_kernel(col_hbm, ones_hbm, zeros_hbm, out_hbm, col_v, ones_v, zb_v,
                   deg_s):
        cid = lax.axis_index("c")
        sid = lax.axis_index("s")
        wid = sid * NC + cid
        pltpu.sync_copy(zeros_hbm, zb_v)
        pltpu.sync_copy(zb_v, deg_s.at[pl.ds(sid * rows_t, rows_t)])
        pltpu.sync_copy(ones_hbm, ones_v)
        pltpu.sync_copy(col_hbm.at[wid], col_v)
        plsc.subcore_barrier()

        def body(j, carry):
            pltpu.sync_copy(ones_v, deg_s.at[col_v.at[j]], add=True)
            return carry

        lax.fori_loop(0, ch, body, 0)
        plsc.subcore_barrier()
        pltpu.sync_copy(deg_s.at[pl.ds(sid * rows_t, rows_t)],
                        out_hbm.at[pl.ds(cid * np_ + sid * rows_t, rows_t)])

    return deg_kernel


def _make_sc_msgpass(np_, hdim, c0, c1):
    mesh = plsc.VectorSubcoreMesh(core_axis_name="c", subcore_axis_name="s")
    rows_t = np_ // NS
    cmax = max(c0, c1)
    nzc = rows_t // CK

    zplan = [(q * CK, CK) for q in range(rows_t // CK)]
    if rows_t % CK:
        zplan.append((rows_t - rows_t % CK, rows_t % CK))

    @functools.partial(
        pl.kernel,
        out_type=jax.ShapeDtypeStruct((NC * np_, hdim), jnp.float32),
        mesh=mesh,
        scratch_types=[
            pltpu.VMEM(((cmax + 2) * CK,), jnp.int32),
            pltpu.VMEM((cmax, CK), jnp.int32),
            pltpu.VMEM((CK, hdim), jnp.float32),
            pltpu.VMEM((CK, hdim), jnp.float32),
            pltpu.VMEM_SHARED((np_, hdim), jnp.float32),
        ] + [pltpu.SemaphoreType.DMA] * 4,
    )
    def msg_kernel(g_hbm, row0_hbm, col0_hbm, row1_hbm, col1_hbm, zeros_hbm,
                   out_hbm, ridx_v, col_v, rows_a, rows_b, acc_s, *sems):
        rows_v = (rows_a, rows_b)
        gsem = sems[:2]
        ssem = sems[2:]
        cid = lax.axis_index("c")
        sid = lax.axis_index("s")
        pltpu.sync_copy(zeros_hbm, rows_a)
        for off, cnt in zplan:
            pltpu.sync_copy(rows_a.at[pl.ds(0, cnt)],
                            acc_s.at[pl.ds(sid * rows_t + off, cnt)])

        @pl.when(cid == 0)
        def _():
            pltpu.sync_copy(row0_hbm.at[pl.ds(sid * (c0 + 2) * CK, (c0 + 2) * CK)],
                            ridx_v.at[pl.ds(0, (c0 + 2) * CK)])
            pltpu.sync_copy(col0_hbm.at[sid], col_v.at[pl.ds(0, c0)])

        @pl.when(cid == 1)
        def _():
            pltpu.sync_copy(row1_hbm.at[pl.ds(sid * (c1 + 2) * CK, (c1 + 2) * CK)],
                            ridx_v.at[pl.ds(0, (c1 + 2) * CK)])
            pltpu.sync_copy(col1_hbm.at[sid], col_v.at[pl.ds(0, c1)])

        plsc.subcore_barrier()
        nb2 = (c0 // 2) if c0 == c1 else jnp.where(cid == 0, c0 // 2, c1 // 2)

        for b in range(2):
            pltpu.async_copy(g_hbm.at[ridx_v.at[pl.ds(b * CK, CK)]],
                             rows_v[b], gsem[b])

        def body(t, carry):
            c = 2 * t
            for b in range(2):
                pltpu.make_async_copy(g_hbm.at[ridx_v.at[pl.ds(0, CK)]],
                                      rows_v[b], gsem[b]).wait()
                pltpu.async_copy(rows_v[b], acc_s.at[col_v.at[c + b]],
                                 ssem[b], add=True)
            for b in range(2):
                pltpu.make_async_copy(rows_v[b], acc_s.at[col_v.at[c + b]],
                                      ssem[b]).wait()
                pltpu.async_copy(g_hbm.at[ridx_v.at[pl.ds((c + 2 + b) * CK, CK)]],
                                 rows_v[b], gsem[b])
            return carry

        lax.fori_loop(0, nb2, body, 0)
        for b in range(2):
            pltpu.make_async_copy(g_hbm.at[ridx_v.at[pl.ds(0, CK)]],
                                  rows_v[b], gsem[b]).wait()
        plsc.subcore_barrier()
        pltpu.sync_copy(acc_s.at[pl.ds(sid * rows_t, rows_t)],
                        out_hbm.at[pl.ds(cid * np_ + sid * rows_t, rows_t)])

    return msg_kernel


def _tc_pre(xp, w_in, b_in, degp, n_real, bn):
    np_, d = xp.shape
    hdim = w_in.shape[1]
    grid = np_ // bn

    def body(x_r, w_r, b_r, deg_r, h0_r, g_r, dis_r):
        i = pl.program_id(0)
        h0 = jnp.dot(x_r[...], w_r[...], preferred_element_type=jnp.float32)
        h0 = jnp.maximum(h0 + b_r[...], 0.0)
        rowid = i * bn + lax.broadcasted_iota(jnp.int32, (bn, 1), 0)
        dis = jnp.where(rowid < n_real, lax.rsqrt(deg_r[...]), 0.0)
        dis2 = jnp.broadcast_to(dis, (bn, hdim))
        h0_r[...] = h0
        dis_r[...] = dis2
        g_r[...] = h0 * dis2

    out_shapes = [jax.ShapeDtypeStruct((np_, hdim), jnp.float32)] * 3
    return pl.pallas_call(
        body,
        grid=(grid,),
        in_specs=[
            pl.BlockSpec((bn, d), lambda i: (i, 0)),
            pl.BlockSpec((d, hdim), lambda i: (0, 0)),
            pl.BlockSpec((1, hdim), lambda i: (0, 0)),
            pl.BlockSpec((bn, 1), lambda i: (i, 0)),
        ],
        out_specs=[pl.BlockSpec((bn, hdim), lambda i: (i, 0))] * 3,
        out_shape=out_shapes,
    )(xp, w_in, b_in.reshape(1, hdim), degp)


def _tc_update(agg2, g, h0, dis2, w, alpha, beta, bn):
    np_, hdim = g.shape
    grid = np_ // bn

    def body(a_r, g_r, h0_r, dis_r, w_r, out_r):
        a = a_r[0] + a_r[1]
        aggf = dis_r[...] * (a + g_r[...])
        z = (1.0 - alpha) * aggf + alpha * h0_r[...]
        zw = jnp.dot(z, w_r[...], preferred_element_type=jnp.float32)
        z = (1.0 - beta) * z + beta * zw
        out_r[...] = dis_r[...] * jnp.maximum(z, 0.0)

    return pl.pallas_call(
        body,
        grid=(grid,),
        in_specs=[
            pl.BlockSpec((NC, bn, hdim), lambda i: (0, i, 0)),
            pl.BlockSpec((bn, hdim), lambda i: (i, 0)),
            pl.BlockSpec((bn, hdim), lambda i: (i, 0)),
            pl.BlockSpec((bn, hdim), lambda i: (i, 0)),
            pl.BlockSpec((hdim, hdim), lambda i: (0, 0)),
        ],
        out_specs=pl.BlockSpec((bn, hdim), lambda i: (i, 0)),
        out_shape=jax.ShapeDtypeStruct((np_, hdim), jnp.float32),
    )(agg2, g, h0, dis2, w)


def _tc_final(agg2, g, h0, dis2, w, w_out, b_out, alpha, beta, bn):
    np_, hdim = g.shape
    odim = w_out.shape[1]
    grid = np_ // bn

    def body(a_r, g_r, h0_r, dis_r, w_r, wo_r, bo_r, out_r):
        a = a_r[0] + a_r[1]
        aggf = dis_r[...] * (a + g_r[...])
        z = (1.0 - alpha) * aggf + alpha * h0_r[...]
        zw = jnp.dot(z, w_r[...], preferred_element_type=jnp.float32)
        z = (1.0 - beta) * z + beta * zw
        h = jnp.maximum(z, 0.0)
        out_r[...] = jnp.dot(h, wo_r[...],
                             preferred_element_type=jnp.float32) + bo_r[...]

    return pl.pallas_call(
        body,
        grid=(grid,),
        in_specs=[
            pl.BlockSpec((NC, bn, hdim), lambda i: (0, i, 0)),
            pl.BlockSpec((bn, hdim), lambda i: (i, 0)),
            pl.BlockSpec((bn, hdim), lambda i: (i, 0)),
            pl.BlockSpec((bn, hdim), lambda i: (i, 0)),
            pl.BlockSpec((hdim, hdim), lambda i: (0, 0)),
            pl.BlockSpec((hdim, odim), lambda i: (0, 0)),
            pl.BlockSpec((1, odim), lambda i: (0, 0)),
        ],
        out_specs=pl.BlockSpec((bn, odim), lambda i: (i, 0)),
        out_shape=jax.ShapeDtypeStruct((np_, odim), jnp.float32),
    )(agg2, g, h0, dis2, w, w_out, b_out.reshape(1, odim))


def kernel(x, edge_index, W_in, b_in, Ws, W_out, b_out):
    n, d = x.shape
    hdim = W_in.shape[1]
    nlayers = Ws.shape[0]
    e = edge_index.shape[1]

    np_ = _pad_up(n, 128)
    npd = _pad_up(n, 2048)
    bn = 512 if np_ % 512 == 0 else 128
    nprows = np_ - n

    ctot = -(-e // CK)
    c0 = max(2, 2 * int(round(ctot * SPLIT0 / (2 * NS))))
    c1 = max(2, 2 * (-(-(ctot - NS * c0) // (2 * NS))))
    pad = NS * (c0 + c1) * CK - e

    row = edge_index[0].astype(jnp.int32)
    col = edge_index[1].astype(jnp.int32)
    dummy = n + (jnp.arange(pad, dtype=jnp.int32) % nprows)
    row_f = jnp.concatenate([row, dummy])
    col_f = jnp.concatenate([col, dummy])
    sz0 = NS * c0 * CK
    dummy2 = jnp.broadcast_to(
        n + (jnp.arange(2 * CK, dtype=jnp.int32) % nprows).reshape(1, 2, CK),
        (NS, 2, CK))
    row0 = jnp.concatenate(
        [row_f[:sz0].reshape(NS, c0, CK), dummy2],
        axis=1).reshape(NS * (c0 + 2) * CK)
    col0 = col_f[:sz0].reshape(NS, c0, CK)
    row1 = jnp.concatenate(
        [row_f[sz0:].reshape(NS, c1, CK), dummy2],
        axis=1).reshape(NS * (c1 + 2) * CK)
    col1 = col_f[sz0:].reshape(NS, c1, CK)
    xp = jnp.zeros((np_, d), jnp.float32).at[:n].set(x)
    ones_ck = jnp.ones((CK,), jnp.float32)
    zeros_1d = jnp.zeros((npd // NS,), jnp.float32)
    zeros_g = jnp.zeros((CK, hdim), jnp.float32)

    chd = -(-ctot // NW)
    padd = NW * chd * CK - e
    dummyd = n + (jnp.arange(padd, dtype=jnp.int32) % nprows)
    col_d = jnp.concatenate([col, dummyd]).reshape(NW, chd, CK)
    deg2 = _make_sc_deg(npd, chd)(col_d, ones_ck, zeros_1d).reshape(NC, npd)
    degp = (deg2[0, :np_] + deg2[1, :np_] + 1.0).reshape(np_, 1)

    h0, g, dis2 = _tc_pre(xp, W_in, b_in, degp, n, bn)

    msg = _make_sc_msgpass(np_, hdim, c0, c1)
    for i in range(nlayers):
        beta = float(np.log(THETA_C / (i + 1) + 1.0))
        agg2 = msg(g, row0, col0, row1, col1,
                   zeros_g).reshape(NC, np_, hdim)
        if i + 1 < nlayers:
            g = _tc_update(agg2, g, h0, dis2, Ws[i], ALPHA_C, beta, bn)
        else:
            out = _tc_final(agg2, g, h0, dis2, Ws[i], W_out, b_out,
                            ALPHA_C, beta, bn)
    return out[:n]


# speedup vs baseline: 1.1953x; 1.1953x over previous
import functools

import numpy as np
import jax
import jax.numpy as jnp
from jax import lax
from jax.experimental import pallas as pl
from jax.experimental.pallas import tpu as pltpu
from jax.experimental.pallas import tpu_sc as plsc

NC = 2
NS = 16
NW = NC * NS
CK = 96

ALPHA_C = 0.1
THETA_C = 0.5
SPLIT0 = 0.5


def _pad_up(v, m):
    return (v + m - 1) // m * m


def _make_sc_deg(np_, ch):
    mesh = plsc.VectorSubcoreMesh(core_axis_name="c", subcore_axis_name="s")
    rows_t = np_ // NS

    @functools.partial(
        pl.kernel,
        out_type=jax.ShapeDtypeStruct((NC * np_,), jnp.float32),
        mesh=mesh,
        scratch_types=[
            pltpu.VMEM((ch, CK), jnp.int32),
            pltpu.VMEM((CK,), jnp.float32),
            pltpu.VMEM((rows_t,), jnp.float32),
            pltpu.VMEM_SHARED((np_,), jnp.float32),
        ],
    )
    def deg_kernel(col_hbm, ones_hbm, zeros_hbm, out_hbm, col_v, ones_v, zb_v,
                   deg_s):
        cid = lax.axis_index("c")
        sid = lax.axis_index("s")
        wid = sid * NC + cid
        pltpu.sync_copy(zeros_hbm, zb_v)
        pltpu.sync_copy(zb_v, deg_s.at[pl.ds(sid * rows_t, rows_t)])
        pltpu.sync_copy(ones_hbm, ones_v)
        pltpu.sync_copy(col_hbm.at[wid], col_v)
        plsc.subcore_barrier()

        def body(j, carry):
            pltpu.sync_copy(ones_v, deg_s.at[col_v.at[j]], add=True)
            return carry

        lax.fori_loop(0, ch, body, 0)
        plsc.subcore_barrier()
        pltpu.sync_copy(deg_s.at[pl.ds(sid * rows_t, rows_t)],
                        out_hbm.at[pl.ds(cid * np_ + sid * rows_t, rows_t)])

    return deg_kernel


def _make_sc_msgpass(np_, hdim, c0, c1):
    mesh = plsc.VectorSubcoreMesh(core_axis_name="c", subcore_axis_name="s")
    rows_t = np_ // NS
    cmax = max(c0, c1)
    nzc = rows_t // CK

    zplan = [(q * CK, CK) for q in range(rows_t // CK)]
    if rows_t % CK:
        zplan.append((rows_t - rows_t % CK, rows_t % CK))

    @functools.partial(
        pl.kernel,
        out_type=jax.ShapeDtypeStruct((NC * np_, hdim), jnp.float32),
        mesh=mesh,
        scratch_types=[
            pltpu.VMEM(((cmax + 2) * CK,), jnp.int32),
            pltpu.VMEM((cmax, CK), jnp.int32),
            pltpu.VMEM((CK, hdim), jnp.float32),
            pltpu.VMEM((CK, hdim), jnp.float32),
            pltpu.VMEM_SHARED((np_, hdim), jnp.float32),
        ] + [pltpu.SemaphoreType.DMA] * 4,
    )
    def msg_kernel(g_hbm, row0_hbm, col0_hbm, row1_hbm, col1_hbm, zeros_hbm,
                   out_hbm, ridx_v, col_v, rows_a, rows_b, acc_s, *sems):
        rows_v = (rows_a, rows_b)
        gsem = sems[:2]
        ssem = sems[2:]
        cid = lax.axis_index("c")
        sid = lax.axis_index("s")
        pltpu.sync_copy(zeros_hbm, rows_a)
        for off, cnt in zplan:
            pltpu.sync_copy(rows_a.at[pl.ds(0, cnt)],
                            acc_s.at[pl.ds(sid * rows_t + off, cnt)])

        @pl.when(cid == 0)
        def _():
            pltpu.sync_copy(row0_hbm.at[pl.ds(sid * (c0 + 2) * CK, (c0 + 2) * CK)],
                            ridx_v.at[pl.ds(0, (c0 + 2) * CK)])
            pltpu.sync_copy(col0_hbm.at[sid], col_v.at[pl.ds(0, c0)])

        @pl.when(cid == 1)
        def _():
            pltpu.sync_copy(row1_hbm.at[pl.ds(sid * (c1 + 2) * CK, (c1 + 2) * CK)],
                            ridx_v.at[pl.ds(0, (c1 + 2) * CK)])
            pltpu.sync_copy(col1_hbm.at[sid], col_v.at[pl.ds(0, c1)])

        plsc.subcore_barrier()
        nb2 = (c0 // 2) if c0 == c1 else jnp.where(cid == 0, c0 // 2, c1 // 2)

        for b in range(2):
            pltpu.async_copy(g_hbm.at[ridx_v.at[pl.ds(b * CK, CK)]],
                             rows_v[b], gsem[b])

        def body(t, carry):
            for b in range(2):
                c = 2 * t + b
                pltpu.make_async_copy(g_hbm.at[ridx_v.at[pl.ds(0, CK)]],
                                      rows_v[b], gsem[b]).wait()
                pltpu.async_copy(rows_v[b], acc_s.at[col_v.at[c]],
                                 ssem[b], add=True)
                pltpu.make_async_copy(rows_v[b], acc_s.at[col_v.at[c]],
                                      ssem[b]).wait()
                pltpu.async_copy(g_hbm.at[ridx_v.at[pl.ds((c + 2) * CK, CK)]],
                                 rows_v[b], gsem[b])
            return carry

        lax.fori_loop(0, nb2, body, 0)
        for b in range(2):
            pltpu.make_async_copy(g_hbm.at[ridx_v.at[pl.ds(0, CK)]],
                                  rows_v[b], gsem[b]).wait()
        plsc.subcore_barrier()
        pltpu.sync_copy(acc_s.at[pl.ds(sid * rows_t, rows_t)],
                        out_hbm.at[pl.ds(cid * np_ + sid * rows_t, rows_t)])

    return msg_kernel


def _tc_pre(xp, w_in, b_in, degp, n_real, bn):
    np_, d = xp.shape
    hdim = w_in.shape[1]
    grid = np_ // bn

    def body(x_r, w_r, b_r, deg_r, h0_r, g_r):
        i = pl.program_id(0)
        h0 = jnp.dot(x_r[...], w_r[...], preferred_element_type=jnp.float32)
        h0 = jnp.maximum(h0 + b_r[...], 0.0)
        rowid = i * bn + lax.broadcasted_iota(jnp.int32, (bn, 1), 0)
        dis = jnp.where(rowid < n_real, lax.rsqrt(deg_r[...]), 0.0)
        dis2 = jnp.broadcast_to(dis, (bn, hdim))
        h0_r[...] = h0
        g_r[...] = h0 * dis2

    out_shapes = [jax.ShapeDtypeStruct((np_, hdim), jnp.float32)] * 2
    return pl.pallas_call(
        body,
        grid=(grid,),
        in_specs=[
            pl.BlockSpec((bn, d), lambda i: (i, 0)),
            pl.BlockSpec((d, hdim), lambda i: (0, 0)),
            pl.BlockSpec((1, hdim), lambda i: (0, 0)),
            pl.BlockSpec((bn, 1), lambda i: (i, 0)),
        ],
        out_specs=[pl.BlockSpec((bn, hdim), lambda i: (i, 0))] * 2,
        out_shape=out_shapes,
    )(xp, w_in, b_in.reshape(1, hdim), degp)


def _tc_update(agg2, g, h0, degp, w, alpha, beta, bn, n_real):
    np_, hdim = g.shape
    grid = np_ // bn

    def body(a_r, g_r, h0_r, deg_r, w_r, out_r):
        i = pl.program_id(0)
        rowid = i * bn + lax.broadcasted_iota(jnp.int32, (bn, 1), 0)
        dis = jnp.where(rowid < n_real, lax.rsqrt(deg_r[...]), 0.0)
        dis2 = jnp.broadcast_to(dis, (bn, hdim))
        a = a_r[0] + a_r[1]
        aggf = dis2 * (a + g_r[...])
        z = (1.0 - alpha) * aggf + alpha * h0_r[...]
        zw = jnp.dot(z, w_r[...], preferred_element_type=jnp.float32)
        z = (1.0 - beta) * z + beta * zw
        out_r[...] = dis2 * jnp.maximum(z, 0.0)

    return pl.pallas_call(
        body,
        grid=(grid,),
        in_specs=[
            pl.BlockSpec((NC, bn, hdim), lambda i: (0, i, 0)),
            pl.BlockSpec((bn, hdim), lambda i: (i, 0)),
            pl.BlockSpec((bn, hdim), lambda i: (i, 0)),
            pl.BlockSpec((bn, 1), lambda i: (i, 0)),
            pl.BlockSpec((hdim, hdim), lambda i: (0, 0)),
        ],
        out_specs=pl.BlockSpec((bn, hdim), lambda i: (i, 0)),
        out_shape=jax.ShapeDtypeStruct((np_, hdim), jnp.float32),
    )(agg2, g, h0, degp, w)


def _tc_final(agg2, g, h0, degp, w, w_out, b_out, alpha, beta, bn, n_real):
    np_, hdim = g.shape
    odim = w_out.shape[1]
    grid = np_ // bn

    def body(a_r, g_r, h0_r, deg_r, w_r, wo_r, bo_r, out_r):
        i = pl.program_id(0)
        rowid = i * bn + lax.broadcasted_iota(jnp.int32, (bn, 1), 0)
        dis = jnp.where(rowid < n_real, lax.rsqrt(deg_r[...]), 0.0)
        dis2 = jnp.broadcast_to(dis, (bn, hdim))
        a = a_r[0] + a_r[1]
        aggf = dis2 * (a + g_r[...])
        z = (1.0 - alpha) * aggf + alpha * h0_r[...]
        zw = jnp.dot(z, w_r[...], preferred_element_type=jnp.float32)
        z = (1.0 - beta) * z + beta * zw
        h = jnp.maximum(z, 0.0)
        out_r[...] = jnp.dot(h, wo_r[...],
                             preferred_element_type=jnp.float32) + bo_r[...]

    return pl.pallas_call(
        body,
        grid=(grid,),
        in_specs=[
            pl.BlockSpec((NC, bn, hdim), lambda i: (0, i, 0)),
            pl.BlockSpec((bn, hdim), lambda i: (i, 0)),
            pl.BlockSpec((bn, hdim), lambda i: (i, 0)),
            pl.BlockSpec((bn, 1), lambda i: (i, 0)),
            pl.BlockSpec((hdim, hdim), lambda i: (0, 0)),
            pl.BlockSpec((hdim, odim), lambda i: (0, 0)),
            pl.BlockSpec((1, odim), lambda i: (0, 0)),
        ],
        out_specs=pl.BlockSpec((bn, odim), lambda i: (i, 0)),
        out_shape=jax.ShapeDtypeStruct((np_, odim), jnp.float32),
    )(agg2, g, h0, degp, w, w_out, b_out.reshape(1, odim))


def kernel(x, edge_index, W_in, b_in, Ws, W_out, b_out):
    n, d = x.shape
    hdim = W_in.shape[1]
    nlayers = Ws.shape[0]
    e = edge_index.shape[1]

    np_ = _pad_up(n, 128)
    npd = _pad_up(n, 2048)
    bn = 512 if np_ % 512 == 0 else 128
    nprows = np_ - n

    ctot = -(-e // CK)
    c0 = max(2, 2 * int(round(ctot * SPLIT0 / (2 * NS))))
    c1 = max(2, 2 * (-(-(ctot - NS * c0) // (2 * NS))))
    pad = NS * (c0 + c1) * CK - e

    row = edge_index[0].astype(jnp.int32)
    col = edge_index[1].astype(jnp.int32)
    dummy = n + (jnp.arange(pad, dtype=jnp.int32) % nprows)
    row_f = jnp.concatenate([row, dummy])
    col_f = jnp.concatenate([col, dummy])
    sz0 = NS * c0 * CK
    dummy2 = jnp.broadcast_to(
        n + (jnp.arange(2 * CK, dtype=jnp.int32) % nprows).reshape(1, 2, CK),
        (NS, 2, CK))
    row0 = jnp.concatenate(
        [row_f[:sz0].reshape(NS, c0, CK), dummy2],
        axis=1).reshape(NS * (c0 + 2) * CK)
    col0 = col_f[:sz0].reshape(NS, c0, CK)
    row1 = jnp.concatenate(
        [row_f[sz0:].reshape(NS, c1, CK), dummy2],
        axis=1).reshape(NS * (c1 + 2) * CK)
    col1 = col_f[sz0:].reshape(NS, c1, CK)
    xp = jnp.zeros((np_, d), jnp.float32).at[:n].set(x)
    ones_ck = jnp.ones((CK,), jnp.float32)
    zeros_1d = jnp.zeros((npd // NS,), jnp.float32)
    zeros_g = jnp.zeros((CK, hdim), jnp.float32)

    chd = -(-ctot // NW)
    padd = NW * chd * CK - e
    dummyd = n + (jnp.arange(padd, dtype=jnp.int32) % nprows)
    col_d = jnp.concatenate([col, dummyd]).reshape(NW, chd, CK)
    deg2 = _make_sc_deg(npd, chd)(col_d, ones_ck, zeros_1d).reshape(NC, npd)
    degp = (deg2[0, :np_] + deg2[1, :np_] + 1.0).reshape(np_, 1)

    h0, g = _tc_pre(xp, W_in, b_in, degp, n, bn)

    msg = _make_sc_msgpass(np_, hdim, c0, c1)
    for i in range(nlayers):
        beta = float(np.log(THETA_C / (i + 1) + 1.0))
        agg2 = msg(g, row0, col0, row1, col1,
                   zeros_g).reshape(NC, np_, hdim)
        if i + 1 < nlayers:
            g = _tc_update(agg2, g, h0, degp, Ws[i], ALPHA_C, beta, bn, n)
        else:
            out = _tc_final(agg2, g, h0, degp, Ws[i], W_out, b_out,
                            ALPHA_C, beta, bn, n)
    return out[:n]


# speedup vs baseline: 1.5226x; 1.2739x over previous
import functools

import numpy as np
import jax
import jax.numpy as jnp
from jax import lax
from jax.experimental import pallas as pl
from jax.experimental.pallas import tpu as pltpu
from jax.experimental.pallas import tpu_sc as plsc

NC = 2
NS = 16
NW = NC * NS
CK = 96

ALPHA_C = 0.1
THETA_C = 0.5
SPLIT0 = 0.5


def _pad_up(v, m):
    return (v + m - 1) // m * m


def _make_sc_deg(np_, ch):
    mesh = plsc.VectorSubcoreMesh(core_axis_name="c", subcore_axis_name="s")
    rows_t = np_ // NS

    @functools.partial(
        pl.kernel,
        out_type=jax.ShapeDtypeStruct((NC * np_,), jnp.float32),
        mesh=mesh,
        scratch_types=[
            pltpu.VMEM((ch, CK), jnp.int32),
            pltpu.VMEM((CK,), jnp.float32),
            pltpu.VMEM((rows_t,), jnp.float32),
            pltpu.VMEM_SHARED((np_,), jnp.float32),
        ],
    )
    def deg_kernel(col_hbm, ones_hbm, zeros_hbm, out_hbm, col_v, ones_v, zb_v,
                   deg_s):
        cid = lax.axis_index("c")
        sid = lax.axis_index("s")
        wid = sid * NC + cid
        pltpu.sync_copy(zeros_hbm, zb_v)
        pltpu.sync_copy(zb_v, deg_s.at[pl.ds(sid * rows_t, rows_t)])
        pltpu.sync_copy(ones_hbm, ones_v)
        pltpu.sync_copy(col_hbm.at[wid], col_v)
        plsc.subcore_barrier()

        def body(j, carry):
            pltpu.sync_copy(ones_v, deg_s.at[col_v.at[j]], add=True)
            return carry

        lax.fori_loop(0, ch, body, 0)
        plsc.subcore_barrier()
        pltpu.sync_copy(deg_s.at[pl.ds(sid * rows_t, rows_t)],
                        out_hbm.at[pl.ds(cid * np_ + sid * rows_t, rows_t)])

    return deg_kernel


def _make_sc_msgpass(np_, hdim, c0, c1):
    mesh = plsc.VectorSubcoreMesh(core_axis_name="c", subcore_axis_name="s")
    rows_t = np_ // NS
    cmax = max(c0, c1)
    nzc = rows_t // CK

    zplan = [(q * CK, CK) for q in range(rows_t // CK)]
    if rows_t % CK:
        zplan.append((rows_t - rows_t % CK, rows_t % CK))

    @functools.partial(
        pl.kernel,
        out_type=jax.ShapeDtypeStruct((NC * np_, hdim), jnp.float32),
        mesh=mesh,
        scratch_types=[
            pltpu.VMEM(((cmax + 2) * CK,), jnp.int32),
            pltpu.VMEM((cmax, CK), jnp.int32),
            pltpu.VMEM((CK, hdim), jnp.float32),
            pltpu.VMEM((CK, hdim), jnp.float32),
            pltpu.VMEM_SHARED((np_, hdim), jnp.float32),
        ] + [pltpu.SemaphoreType.DMA] * 4,
    )
    def msg_kernel(g_hbm, row0_hbm, col0_hbm, row1_hbm, col1_hbm, zeros_hbm,
                   out_hbm, ridx_v, col_v, rows_a, rows_b, acc_s, *sems):
        rows_v = (rows_a, rows_b)
        gsem = sems[:2]
        ssem = sems[2:]
        cid = lax.axis_index("c")
        sid = lax.axis_index("s")
        pltpu.sync_copy(zeros_hbm, rows_a)
        for off, cnt in zplan:
            pltpu.sync_copy(rows_a.at[pl.ds(0, cnt)],
                            acc_s.at[pl.ds(sid * rows_t + off, cnt)])

        @pl.when(cid == 0)
        def _():
            pltpu.sync_copy(row0_hbm.at[pl.ds(sid * (c0 + 2) * CK, (c0 + 2) * CK)],
                            ridx_v.at[pl.ds(0, (c0 + 2) * CK)])
            pltpu.sync_copy(col0_hbm.at[sid], col_v.at[pl.ds(0, c0)])

        @pl.when(cid == 1)
        def _():
            pltpu.sync_copy(row1_hbm.at[pl.ds(sid * (c1 + 2) * CK, (c1 + 2) * CK)],
                            ridx_v.at[pl.ds(0, (c1 + 2) * CK)])
            pltpu.sync_copy(col1_hbm.at[sid], col_v.at[pl.ds(0, c1)])

        plsc.subcore_barrier()
        nb2 = (c0 // 2) if c0 == c1 else jnp.where(cid == 0, c0 // 2, c1 // 2)

        for b in range(2):
            pltpu.async_copy(g_hbm.at[ridx_v.at[pl.ds(b * CK, CK)]],
                             rows_v[b], gsem[b])

        def body(t, carry):
            for b in range(2):
                c = 2 * t + b
                pltpu.make_async_copy(g_hbm.at[ridx_v.at[pl.ds(0, CK)]],
                                      rows_v[b], gsem[b]).wait()
                pltpu.async_copy(rows_v[b], acc_s.at[col_v.at[c]],
                                 ssem[b], add=True)
                pltpu.make_async_copy(rows_v[b], acc_s.at[col_v.at[c]],
                                      ssem[b]).wait()
                pltpu.async_copy(g_hbm.at[ridx_v.at[pl.ds((c + 2) * CK, CK)]],
                                 rows_v[b], gsem[b])
            return carry

        lax.fori_loop(0, nb2, body, 0)
        for b in range(2):
            pltpu.make_async_copy(g_hbm.at[ridx_v.at[pl.ds(0, CK)]],
                                  rows_v[b], gsem[b]).wait()
        plsc.subcore_barrier()
        pltpu.sync_copy(acc_s.at[pl.ds(sid * rows_t, rows_t)],
                        out_hbm.at[pl.ds(cid * np_ + sid * rows_t, rows_t)])

    return msg_kernel


def _tc_pre(xp, w_in, b_in, degp, n_real, bn):
    np_, d = xp.shape
    hdim = w_in.shape[1]
    grid = np_ // bn

    def body(x_r, w_r, b_r, deg_r, h0_r, g_r):
        i = pl.program_id(0)
        h0 = jnp.dot(x_r[...], w_r[...], preferred_element_type=jnp.float32)
        h0 = jnp.maximum(h0 + b_r[...], 0.0)
        rowid = i * bn + lax.broadcasted_iota(jnp.int32, (bn, 1), 0)
        dis = jnp.where(rowid < n_real, lax.rsqrt(deg_r[...]), 0.0)
        dis2 = jnp.broadcast_to(dis, (bn, hdim))
        h0_r[...] = h0
        g_r[...] = h0 * dis2

    out_shapes = [jax.ShapeDtypeStruct((np_, hdim), jnp.float32)] * 2
    return pl.pallas_call(
        body,
        grid=(grid,),
        in_specs=[
            pl.BlockSpec((bn, d), lambda i: (i, 0)),
            pl.BlockSpec((d, hdim), lambda i: (0, 0)),
            pl.BlockSpec((1, hdim), lambda i: (0, 0)),
            pl.BlockSpec((bn, 1), lambda i: (i, 0)),
        ],
        out_specs=[pl.BlockSpec((bn, hdim), lambda i: (i, 0))] * 2,
        out_shape=out_shapes,
    )(xp, w_in, b_in.reshape(1, hdim), degp)


def _tc_update(agg2, g, h0, degp, w, alpha, beta, bn, n_real):
    np_, hdim = g.shape
    grid = np_ // bn

    def body(a_r, g_r, h0_r, deg_r, w_r, out_r):
        i = pl.program_id(0)
        rowid = i * bn + lax.broadcasted_iota(jnp.int32, (bn, 1), 0)
        dis = jnp.where(rowid < n_real, lax.rsqrt(deg_r[...]), 0.0)
        dis2 = jnp.broadcast_to(dis, (bn, hdim))
        a = a_r[0] + a_r[1]
        aggf = dis2 * (a + g_r[...])
        z = (1.0 - alpha) * aggf + alpha * h0_r[...]
        zw = jnp.dot(z, w_r[...], preferred_element_type=jnp.float32)
        z = (1.0 - beta) * z + beta * zw
        out_r[...] = dis2 * jnp.maximum(z, 0.0)

    return pl.pallas_call(
        body,
        grid=(grid,),
        in_specs=[
            pl.BlockSpec((NC, bn, hdim), lambda i: (0, i, 0)),
            pl.BlockSpec((bn, hdim), lambda i: (i, 0)),
            pl.BlockSpec((bn, hdim), lambda i: (i, 0)),
            pl.BlockSpec((bn, 1), lambda i: (i, 0)),
            pl.BlockSpec((hdim, hdim), lambda i: (0, 0)),
        ],
        out_specs=pl.BlockSpec((bn, hdim), lambda i: (i, 0)),
        out_shape=jax.ShapeDtypeStruct((np_, hdim), jnp.float32),
    )(agg2, g, h0, degp, w)


def _tc_final(agg2, g, h0, degp, w, w_out, b_out, alpha, beta, bn, n_real):
    np_, hdim = g.shape
    odim = w_out.shape[1]
    grid = np_ // bn

    def body(a_r, g_r, h0_r, deg_r, w_r, wo_r, bo_r, out_r):
        i = pl.program_id(0)
        rowid = i * bn + lax.broadcasted_iota(jnp.int32, (bn, 1), 0)
        dis = jnp.where(rowid < n_real, lax.rsqrt(deg_r[...]), 0.0)
        dis2 = jnp.broadcast_to(dis, (bn, hdim))
        a = a_r[0] + a_r[1]
        aggf = dis2 * (a + g_r[...])
        z = (1.0 - alpha) * aggf + alpha * h0_r[...]
        zw = jnp.dot(z, w_r[...], preferred_element_type=jnp.float32)
        z = (1.0 - beta) * z + beta * zw
        h = jnp.maximum(z, 0.0)
        out_r[...] = jnp.dot(h, wo_r[...],
                             preferred_element_type=jnp.float32) + bo_r[...]

    return pl.pallas_call(
        body,
        grid=(grid,),
        in_specs=[
            pl.BlockSpec((NC, bn, hdim), lambda i: (0, i, 0)),
            pl.BlockSpec((bn, hdim), lambda i: (i, 0)),
            pl.BlockSpec((bn, hdim), lambda i: (i, 0)),
            pl.BlockSpec((bn, 1), lambda i: (i, 0)),
            pl.BlockSpec((hdim, hdim), lambda i: (0, 0)),
            pl.BlockSpec((hdim, odim), lambda i: (0, 0)),
            pl.BlockSpec((1, odim), lambda i: (0, 0)),
        ],
        out_specs=pl.BlockSpec((bn, odim), lambda i: (i, 0)),
        out_shape=jax.ShapeDtypeStruct((np_, odim), jnp.float32),
    )(agg2, g, h0, degp, w, w_out, b_out.reshape(1, odim))


def kernel(x, edge_index, W_in, b_in, Ws, W_out, b_out):
    n, d = x.shape
    hdim = W_in.shape[1]
    nlayers = Ws.shape[0]
    e = edge_index.shape[1]

    np_ = _pad_up(n, 128)
    npd = _pad_up(n, 2048)
    bn = 512 if np_ % 512 == 0 else np_ // NS
    nprows = np_ - n

    ctot = -(-e // CK)
    c0 = max(2, 2 * int(round(ctot * SPLIT0 / (2 * NS))))
    c1 = max(2, 2 * (-(-(ctot - NS * c0) // (2 * NS))))
    pad = NS * (c0 + c1) * CK - e

    row = edge_index[0].astype(jnp.int32)
    col = edge_index[1].astype(jnp.int32)
    dummy = n + (jnp.arange(pad, dtype=jnp.int32) % nprows)
    row_f = jnp.concatenate([row, dummy])
    col_f = jnp.concatenate([col, dummy])
    sz0 = NS * c0 * CK
    dummy2 = jnp.broadcast_to(
        n + (jnp.arange(2 * CK, dtype=jnp.int32) % nprows).reshape(1, 2, CK),
        (NS, 2, CK))
    row0 = jnp.concatenate(
        [row_f[:sz0].reshape(NS, c0, CK), dummy2],
        axis=1).reshape(NS * (c0 + 2) * CK)
    col0 = col_f[:sz0].reshape(NS, c0, CK)
    row1 = jnp.concatenate(
        [row_f[sz0:].reshape(NS, c1, CK), dummy2],
        axis=1).reshape(NS * (c1 + 2) * CK)
    col1 = col_f[sz0:].reshape(NS, c1, CK)
    xp = jnp.zeros((np_, d), jnp.float32).at[:n].set(x)
    ones_ck = jnp.ones((CK,), jnp.float32)
    zeros_1d = jnp.zeros((npd // NS,), jnp.float32)
    zeros_g = jnp.zeros((CK, hdim), jnp.float32)

    chd = -(-ctot // NW)
    padd = NW * chd * CK - e
    dummyd = n + (jnp.arange(padd, dtype=jnp.int32) % nprows)
    col_d = jnp.concatenate([col, dummyd]).reshape(NW, chd, CK)
    deg2 = _make_sc_deg(npd, chd)(col_d, ones_ck, zeros_1d).reshape(NC, npd)
    degp = (deg2[0, :np_] + deg2[1, :np_] + 1.0).reshape(np_, 1)

    h0, g = _tc_pre(xp, W_in, b_in, degp, n, bn)

    msg = _make_sc_msgpass(np_, hdim, c0, c1)
    for i in range(nlayers):
        beta = float(np.log(THETA_C / (i + 1) + 1.0))
        agg2 = msg(g, row0, col0, row1, col1,
                   zeros_g).reshape(NC, np_, hdim)
        if i + 1 < nlayers:
            g = _tc_update(agg2, g, h0, degp, Ws[i], ALPHA_C, beta, bn, n)
        else:
            out = _tc_final(agg2, g, h0, degp, Ws[i], W_out, b_out,
                            ALPHA_C, beta, bn, n)
    return out[:n]


# speedup vs baseline: 1.5820x; 1.0390x over previous
import functools

import numpy as np
import jax
import jax.numpy as jnp
from jax import lax
from jax.experimental import pallas as pl
from jax.experimental.pallas import tpu as pltpu
from jax.experimental.pallas import tpu_sc as plsc

NC = 2
NS = 16
NW = NC * NS
CK = 96

ALPHA_C = 0.1
THETA_C = 0.5
SPLIT0 = 0.5


def _pad_up(v, m):
    return (v + m - 1) // m * m


def _make_sc_deg(np_, ch):
    mesh = plsc.VectorSubcoreMesh(core_axis_name="c", subcore_axis_name="s")
    rows_t = np_ // NS

    @functools.partial(
        pl.kernel,
        out_type=jax.ShapeDtypeStruct((NC * np_,), jnp.float32),
        mesh=mesh,
        scratch_types=[
            pltpu.VMEM((ch, CK), jnp.int32),
            pltpu.VMEM((CK,), jnp.float32),
            pltpu.VMEM((rows_t,), jnp.float32),
            pltpu.VMEM_SHARED((np_,), jnp.float32),
        ],
    )
    def deg_kernel(col_hbm, ones_hbm, zeros_hbm, out_hbm, col_v, ones_v, zb_v,
                   deg_s):
        cid = lax.axis_index("c")
        sid = lax.axis_index("s")
        wid = sid * NC + cid
        pltpu.sync_copy(zeros_hbm, zb_v)
        pltpu.sync_copy(zb_v, deg_s.at[pl.ds(sid * rows_t, rows_t)])
        pltpu.sync_copy(ones_hbm, ones_v)
        pltpu.sync_copy(col_hbm.at[wid], col_v)
        plsc.subcore_barrier()

        def body(j, carry):
            pltpu.sync_copy(ones_v, deg_s.at[col_v.at[j]], add=True)
            return carry

        lax.fori_loop(0, ch, body, 0)
        plsc.subcore_barrier()
        pltpu.sync_copy(deg_s.at[pl.ds(sid * rows_t, rows_t)],
                        out_hbm.at[pl.ds(cid * np_ + sid * rows_t, rows_t)])

    return deg_kernel


def _make_sc_msgpass(np_, hdim, c0, c1):
    mesh = plsc.VectorSubcoreMesh(core_axis_name="c", subcore_axis_name="s")
    rows_t = np_ // NS
    cmax = max(c0, c1)
    nzc = rows_t // CK

    zplan = [(q * CK, CK) for q in range(rows_t // CK)]
    if rows_t % CK:
        zplan.append((rows_t - rows_t % CK, rows_t % CK))

    @functools.partial(
        pl.kernel,
        out_type=jax.ShapeDtypeStruct((NC * np_, hdim), jnp.float32),
        mesh=mesh,
        scratch_types=[
            pltpu.VMEM(((cmax + 2) * CK,), jnp.int32),
            pltpu.VMEM((cmax, CK), jnp.int32),
            pltpu.VMEM((CK, hdim), jnp.float32),
            pltpu.VMEM((CK, hdim), jnp.float32),
            pltpu.VMEM_SHARED((np_, hdim), jnp.float32),
        ] + [pltpu.SemaphoreType.DMA] * 4,
    )
    def msg_kernel(g_hbm, row0_hbm, col0_hbm, row1_hbm, col1_hbm, zeros_hbm,
                   out_hbm, ridx_v, col_v, rows_a, rows_b, acc_s, *sems):
        rows_v = (rows_a, rows_b)
        gsem = sems[:2]
        ssem = sems[2:]
        cid = lax.axis_index("c")
        sid = lax.axis_index("s")
        pltpu.sync_copy(zeros_hbm, rows_a)
        for off, cnt in zplan:
            pltpu.sync_copy(rows_a.at[pl.ds(0, cnt)],
                            acc_s.at[pl.ds(sid * rows_t + off, cnt)])

        @pl.when(cid == 0)
        def _():
            pltpu.sync_copy(row0_hbm.at[pl.ds(sid * (c0 + 2) * CK, (c0 + 2) * CK)],
                            ridx_v.at[pl.ds(0, (c0 + 2) * CK)])
            pltpu.sync_copy(col0_hbm.at[sid], col_v.at[pl.ds(0, c0)])

        @pl.when(cid == 1)
        def _():
            pltpu.sync_copy(row1_hbm.at[pl.ds(sid * (c1 + 2) * CK, (c1 + 2) * CK)],
                            ridx_v.at[pl.ds(0, (c1 + 2) * CK)])
            pltpu.sync_copy(col1_hbm.at[sid], col_v.at[pl.ds(0, c1)])

        plsc.subcore_barrier()
        nb2 = (c0 // 2) if c0 == c1 else jnp.where(cid == 0, c0 // 2, c1 // 2)

        for b in range(2):
            pltpu.async_copy(g_hbm.at[ridx_v.at[pl.ds(b * CK, CK)]],
                             rows_v[b], gsem[b])

        def body(t, carry):
            for b in range(2):
                c = 2 * t + b
                pltpu.make_async_copy(g_hbm.at[ridx_v.at[pl.ds(0, CK)]],
                                      rows_v[b], gsem[b]).wait()
                pltpu.async_copy(rows_v[b], acc_s.at[col_v.at[c]],
                                 ssem[b], add=True)
                pltpu.make_async_copy(rows_v[b], acc_s.at[col_v.at[c]],
                                      ssem[b]).wait()
                pltpu.async_copy(g_hbm.at[ridx_v.at[pl.ds((c + 2) * CK, CK)]],
                                 rows_v[b], gsem[b])
            return carry

        lax.fori_loop(0, nb2, body, 0)
        for b in range(2):
            pltpu.make_async_copy(g_hbm.at[ridx_v.at[pl.ds(0, CK)]],
                                  rows_v[b], gsem[b]).wait()
        plsc.subcore_barrier()
        pltpu.sync_copy(acc_s.at[pl.ds(sid * rows_t, rows_t)],
                        out_hbm.at[pl.ds(cid * np_ + sid * rows_t, rows_t)])

    return msg_kernel


def _tc_pre(xp, w_in, b_in, degp, n_real, bn):
    np_, d = xp.shape
    hdim = w_in.shape[1]
    grid = np_ // bn

    def body(x_r, w_r, b_r, deg_r, h0_r, g_r):
        i = pl.program_id(0)
        h0 = jnp.dot(x_r[...], w_r[...], preferred_element_type=jnp.float32)
        h0 = jnp.maximum(h0 + b_r[...], 0.0)
        rowid = i * bn + lax.broadcasted_iota(jnp.int32, (bn, 1), 0)
        dis = jnp.where(rowid < n_real, lax.rsqrt(deg_r[...]), 0.0)
        dis2 = jnp.broadcast_to(dis, (bn, hdim))
        h0_r[...] = h0
        g_r[...] = h0 * dis2

    out_shapes = [jax.ShapeDtypeStruct((np_, hdim), jnp.float32)] * 2
    return pl.pallas_call(
        body,
        grid=(grid,),
        in_specs=[
            pl.BlockSpec((bn, d), lambda i: (i, 0)),
            pl.BlockSpec((d, hdim), lambda i: (0, 0)),
            pl.BlockSpec((1, hdim), lambda i: (0, 0)),
            pl.BlockSpec((bn, 1), lambda i: (i, 0)),
        ],
        out_specs=[pl.BlockSpec((bn, hdim), lambda i: (i, 0))] * 2,
        out_shape=out_shapes,
    )(xp, w_in, b_in.reshape(1, hdim), degp)


def _tc_update(agg2, g, h0, degp, w, alpha, beta, bn, n_real):
    np_, hdim = g.shape
    grid = np_ // bn

    def body(a_r, g_r, h0_r, deg_r, w_r, out_r):
        i = pl.program_id(0)
        rowid = i * bn + lax.broadcasted_iota(jnp.int32, (bn, 1), 0)
        dis = jnp.where(rowid < n_real, lax.rsqrt(deg_r[...]), 0.0)
        dis2 = jnp.broadcast_to(dis, (bn, hdim))
        a = a_r[0] + a_r[1]
        aggf = dis2 * (a + g_r[...])
        z = (1.0 - alpha) * aggf + alpha * h0_r[...]
        zw = jnp.dot(z, w_r[...], preferred_element_type=jnp.float32)
        z = (1.0 - beta) * z + beta * zw
        out_r[...] = dis2 * jnp.maximum(z, 0.0)

    return pl.pallas_call(
        body,
        grid=(grid,),
        in_specs=[
            pl.BlockSpec((NC, bn, hdim), lambda i: (0, i, 0)),
            pl.BlockSpec((bn, hdim), lambda i: (i, 0)),
            pl.BlockSpec((bn, hdim), lambda i: (i, 0)),
            pl.BlockSpec((bn, 1), lambda i: (i, 0)),
            pl.BlockSpec((hdim, hdim), lambda i: (0, 0)),
        ],
        out_specs=pl.BlockSpec((bn, hdim), lambda i: (i, 0)),
        out_shape=jax.ShapeDtypeStruct((np_, hdim), jnp.float32),
    )(agg2, g, h0, degp, w)


def _tc_final(agg2, g, h0, degp, w, w_out, b_out, alpha, beta, bn, n_real):
    np_, hdim = g.shape
    odim = w_out.shape[1]
    grid = np_ // bn

    def body(a_r, g_r, h0_r, deg_r, w_r, wo_r, bo_r, out_r):
        i = pl.program_id(0)
        rowid = i * bn + lax.broadcasted_iota(jnp.int32, (bn, 1), 0)
        dis = jnp.where(rowid < n_real, lax.rsqrt(deg_r[...]), 0.0)
        dis2 = jnp.broadcast_to(dis, (bn, hdim))
        a = a_r[0] + a_r[1]
        aggf = dis2 * (a + g_r[...])
        z = (1.0 - alpha) * aggf + alpha * h0_r[...]
        zw = jnp.dot(z, w_r[...], preferred_element_type=jnp.float32)
        z = (1.0 - beta) * z + beta * zw
        h = jnp.maximum(z, 0.0)
        out_r[...] = jnp.dot(h, wo_r[...],
                             preferred_element_type=jnp.float32) + bo_r[...]

    return pl.pallas_call(
        body,
        grid=(grid,),
        in_specs=[
            pl.BlockSpec((NC, bn, hdim), lambda i: (0, i, 0)),
            pl.BlockSpec((bn, hdim), lambda i: (i, 0)),
            pl.BlockSpec((bn, hdim), lambda i: (i, 0)),
            pl.BlockSpec((bn, 1), lambda i: (i, 0)),
            pl.BlockSpec((hdim, hdim), lambda i: (0, 0)),
            pl.BlockSpec((hdim, odim), lambda i: (0, 0)),
            pl.BlockSpec((1, odim), lambda i: (0, 0)),
        ],
        out_specs=pl.BlockSpec((bn, odim), lambda i: (i, 0)),
        out_shape=jax.ShapeDtypeStruct((np_, odim), jnp.float32),
    )(agg2, g, h0, degp, w, w_out, b_out.reshape(1, odim))


def kernel(x, edge_index, W_in, b_in, Ws, W_out, b_out):
    n, d = x.shape
    hdim = W_in.shape[1]
    nlayers = Ws.shape[0]
    e = edge_index.shape[1]

    np_ = _pad_up(n, 128)
    npd = _pad_up(n, 2048)
    bn = 512 if np_ % 512 == 0 else np_ // 8
    nprows = np_ - n

    ctot = -(-e // CK)
    c0 = max(2, 2 * int(round(ctot * SPLIT0 / (2 * NS))))
    c1 = max(2, 2 * (-(-(ctot - NS * c0) // (2 * NS))))
    pad = NS * (c0 + c1) * CK - e

    row = edge_index[0].astype(jnp.int32)
    col = edge_index[1].astype(jnp.int32)
    dummy = n + (jnp.arange(pad, dtype=jnp.int32) % nprows)
    row_f = jnp.concatenate([row, dummy])
    col_f = jnp.concatenate([col, dummy])
    sz0 = NS * c0 * CK
    dummy2 = jnp.broadcast_to(
        n + (jnp.arange(2 * CK, dtype=jnp.int32) % nprows).reshape(1, 2, CK),
        (NS, 2, CK))
    row0 = jnp.concatenate(
        [row_f[:sz0].reshape(NS, c0, CK), dummy2],
        axis=1).reshape(NS * (c0 + 2) * CK)
    col0 = col_f[:sz0].reshape(NS, c0, CK)
    row1 = jnp.concatenate(
        [row_f[sz0:].reshape(NS, c1, CK), dummy2],
        axis=1).reshape(NS * (c1 + 2) * CK)
    col1 = col_f[sz0:].reshape(NS, c1, CK)
    xp = jnp.zeros((np_, d), jnp.float32).at[:n].set(x)
    ones_ck = jnp.ones((CK,), jnp.float32)
    zeros_1d = jnp.zeros((npd // NS,), jnp.float32)
    zeros_g = jnp.zeros((CK, hdim), jnp.float32)

    chd = -(-ctot // NW)
    padd = NW * chd * CK - e
    dummyd = n + (jnp.arange(padd, dtype=jnp.int32) % nprows)
    col_d = jnp.concatenate([col, dummyd]).reshape(NW, chd, CK)
    deg2 = _make_sc_deg(npd, chd)(col_d, ones_ck, zeros_1d).reshape(NC, npd)
    degp = (deg2[0, :np_] + deg2[1, :np_] + 1.0).reshape(np_, 1)

    h0, g = _tc_pre(xp, W_in, b_in, degp, n, bn)

    msg = _make_sc_msgpass(np_, hdim, c0, c1)
    for i in range(nlayers):
        beta = float(np.log(THETA_C / (i + 1) + 1.0))
        agg2 = msg(g, row0, col0, row1, col1,
                   zeros_g).reshape(NC, np_, hdim)
        if i + 1 < nlayers:
            g = _tc_update(agg2, g, h0, degp, Ws[i], ALPHA_C, beta, bn, n)
        else:
            out = _tc_final(agg2, g, h0, degp, Ws[i], W_out, b_out,
                            ALPHA_C, beta, bn, n)
    return out[:n]
